# baseline (speedup 1.0000x reference)
import jax
import jax.numpy as jnp
from jax import lax
from jax.experimental import pallas as pl
from jax.experimental.pallas import tpu as pltpu
from jax.experimental.pallas import tpu_sc as plsc

_NC = 2
_NS = 16
_NW = _NC * _NS
_LANES = 16


def kernel(x, row_embed):
    B, L, D = x.shape
    table = row_embed[:L].reshape(-1)
    x_flat = x.reshape(-1)

    l_span = L // _NW
    half = l_span // 2
    chunk_w = half * D
    tbuf_w = l_span * D
    n_chunks = 2 * B
    n_vec = chunk_w // _LANES

    mesh = plsc.VectorSubcoreMesh(core_axis_name="c", subcore_axis_name="s")

    @pl.kernel(
        out_type=jax.ShapeDtypeStruct((B * L * D,), x.dtype),
        mesh=mesh,
        scratch_types=[
            pltpu.VMEM((tbuf_w,), jnp.float32),
            pltpu.VMEM((chunk_w,), jnp.float32),
            pltpu.VMEM((chunk_w,), jnp.float32),
            pltpu.SemaphoreType.DMA,
            pltpu.SemaphoreType.DMA,
            pltpu.SemaphoreType.DMA,
            pltpu.SemaphoreType.DMA,
            pltpu.SemaphoreType.DMA,
        ],
    )
    def sc_kernel(x_hbm, row_hbm, o_hbm, tbuf, ob0, ob1, sem_t, sx0, sx1, so0, so1):
        wid = lax.axis_index("s") * _NC + lax.axis_index("c")
        l_base = wid * l_span
        obufs = (ob0, ob1)
        sxs = (sx0, sx1)
        sos = (so0, so1)

        def off(i):
            b, h = i // 2, i % 2
            return (b * L + l_base + h * half) * D

        t_copy = pltpu.async_copy(
            row_hbm.at[pl.ds(l_base * D, tbuf_w)], tbuf, sem_t
        )
        x_copies = {}
        o_copies = {}
        x_copies[0] = pltpu.async_copy(
            x_hbm.at[pl.ds(off(0), chunk_w)], obufs[0], sxs[0]
        )
        for i in range(n_chunks):
            cur = obufs[i % 2]
            if i + 1 < n_chunks:
                if i - 1 >= 0:
                    o_copies[i - 1].wait()
                x_copies[i + 1] = pltpu.async_copy(
                    x_hbm.at[pl.ds(off(i + 1), chunk_w)],
                    obufs[(i + 1) % 2],
                    sxs[(i + 1) % 2],
                )
            x_copies[i].wait()
            if i == 0:
                t_copy.wait()
            toff = (i % 2) * chunk_w

            @pl.loop(0, n_vec)
            def _(k):
                v = tbuf[pl.ds(toff + k * _LANES, _LANES)]
                plsc.addupdate(cur.at[pl.ds(k * _LANES, _LANES)], v)

            o_copies[i] = pltpu.async_copy(
                cur, o_hbm.at[pl.ds(off(i), chunk_w)], sos[i % 2]
            )
        o_copies[n_chunks - 2].wait()
        o_copies[n_chunks - 1].wait()

    return sc_kernel(x_flat, table).reshape(B, L, D)


# speedup vs baseline: 1.3622x; 1.3622x over previous
import jax
import jax.numpy as jnp
from jax import lax
from jax.experimental import pallas as pl
from jax.experimental.pallas import tpu as pltpu
from jax.experimental.pallas import tpu_sc as plsc

_NC = 2
_NS = 16
_NW = _NC * _NS
_LANES = 16
_UNROLL = 16


def kernel(x, row_embed):
    B, L, D = x.shape
    table = row_embed[:L].reshape(-1)
    x_flat = x.reshape(-1)

    l_span = L // _NW
    half = l_span // 2
    chunk_w = half * D
    tbuf_w = l_span * D
    n_chunks = 2 * B
    n_vec = chunk_w // _LANES

    mesh = plsc.VectorSubcoreMesh(core_axis_name="c", subcore_axis_name="s")

    @pl.kernel(
        out_type=jax.ShapeDtypeStruct((B * L * D,), x.dtype),
        mesh=mesh,
        scratch_types=[
            pltpu.VMEM((tbuf_w,), jnp.float32),
            pltpu.VMEM((chunk_w,), jnp.float32),
            pltpu.VMEM((chunk_w,), jnp.float32),
            pltpu.SemaphoreType.DMA,
            pltpu.SemaphoreType.DMA,
            pltpu.SemaphoreType.DMA,
            pltpu.SemaphoreType.DMA,
            pltpu.SemaphoreType.DMA,
        ],
    )
    def sc_kernel(x_hbm, row_hbm, o_hbm, tbuf, ob0, ob1, sem_t, sx0, sx1, so0, so1):
        wid = lax.axis_index("s") * _NC + lax.axis_index("c")
        l_base = wid * l_span
        obufs = (ob0, ob1)
        sxs = (sx0, sx1)
        sos = (so0, so1)

        def off(i):
            b, h = i // 2, i % 2
            return (b * L + l_base + h * half) * D

        t_copy = pltpu.async_copy(
            row_hbm.at[pl.ds(l_base * D, tbuf_w)], tbuf, sem_t
        )
        x_copies = {}
        o_copies = {}
        x_copies[0] = pltpu.async_copy(
            x_hbm.at[pl.ds(off(0), chunk_w)], obufs[0], sxs[0]
        )
        for i in range(n_chunks):
            cur = obufs[i % 2]
            if i + 1 < n_chunks:
                if i - 1 >= 0:
                    o_copies[i - 1].wait()
                x_copies[i + 1] = pltpu.async_copy(
                    x_hbm.at[pl.ds(off(i + 1), chunk_w)],
                    obufs[(i + 1) % 2],
                    sxs[(i + 1) % 2],
                )
            x_copies[i].wait()
            if i == 0:
                t_copy.wait()
            toff = (i % 2) * chunk_w

            @pl.loop(0, chunk_w, step=_UNROLL * _LANES)
            def _(k):
                for u in range(_UNROLL):
                    s = u * _LANES
                    v = tbuf[pl.ds(toff + k + s, _LANES)]
                    plsc.addupdate(cur.at[pl.ds(k + s, _LANES)], v)

            o_copies[i] = pltpu.async_copy(
                cur, o_hbm.at[pl.ds(off(i), chunk_w)], sos[i % 2]
            )
        o_copies[n_chunks - 2].wait()
        o_copies[n_chunks - 1].wait()

    return sc_kernel(x_flat, table).reshape(B, L, D)


# speedup vs baseline: 7.4659x; 5.4806x over previous
import jax
import jax.numpy as jnp
from jax.experimental import pallas as pl
from jax.experimental.pallas import tpu as pltpu

_LB = 2048


def _add_kernel(x_ref, row_ref, o_ref):
    o_ref[0, :, :] = x_ref[0, :, :] + row_ref[:, :]


def kernel(x, row_embed):
    B, L, D = x.shape
    table = row_embed[:L]
    grid = (L // _LB, B)
    return pl.pallas_call(
        _add_kernel,
        grid=grid,
        in_specs=[
            pl.BlockSpec((1, _LB, D), lambda l, b: (b, l, 0)),
            pl.BlockSpec((_LB, D), lambda l, b: (l, 0)),
        ],
        out_specs=pl.BlockSpec((1, _LB, D), lambda l, b: (b, l, 0)),
        out_shape=jax.ShapeDtypeStruct((B, L, D), x.dtype),
        compiler_params=pltpu.CompilerParams(
            dimension_semantics=("parallel", "arbitrary"),
        ),
    )(x, table)
